# baseline (speedup 1.0000x reference)
import functools
import jax
import jax.numpy as jnp
from jax import lax
from jax.experimental import pallas as pl
from jax.experimental.pallas import tpu as pltpu
from jax.experimental.pallas import tpu_sc as plsc

B, F, T = 1024, 512, 100
NLANE = 128
NC, NS = 2, 16
NW = NC * NS
FH = F // 2
BPW = B // (NW // 2)
VPB = FH // 16


def _enc16(xv):
    s = 1.0 / (1.0 + jnp.exp(-xv))
    v = (1.0 - s) * 99.0
    tr = v.astype(jnp.int32)
    frac = v - tr.astype(jnp.float32)
    gt = jnp.where(frac > 0.5, 1, 0).astype(jnp.int32)
    eq = jnp.where(frac == 0.5, 1, 0).astype(jnp.int32)
    up = gt | (eq & (tr & 1))
    lat = tr + up
    lat = jnp.minimum(jnp.maximum(lat, 0), 99)
    val = jnp.where(s > 0.5, 1.0, 0.0).astype(jnp.float32)
    return lat, val


def kernel(x):
    mesh = plsc.VectorSubcoreMesh(core_axis_name="c", subcore_axis_name="s")

    @functools.partial(
        pl.kernel,
        out_type=jax.ShapeDtypeStruct((B, F, NLANE), jnp.float32),
        mesh=mesh,
        scratch_types=[
            pltpu.VMEM((BPW, FH), jnp.float32),
            pltpu.VMEM((2, FH, NLANE), jnp.float32),
            pltpu.VMEM((2, FH), jnp.int32),
            pltpu.SemaphoreType.DMA((2,)),
        ],
        compiler_params=pltpu.CompilerParams(
            use_tc_tiling_on_sc=True, needs_layout_passes=False),
    )
    def enc(z_hbm, x_hbm, y_hbm, xbuf, rows, idxs, sems):
        wid = lax.axis_index("s") * NC + lax.axis_index("c")
        base = (wid // 2) * BPW
        f0 = (wid % 2) * FH
        pltpu.sync_copy(x_hbm.at[pl.ds(base, BPW), pl.ds(f0, FH)], xbuf)
        pltpu.sync_copy(z_hbm, rows.at[0])
        pltpu.sync_copy(z_hbm, rows.at[1])
        fidx0 = lax.broadcasted_iota(jnp.int32, (16,), 0)
        zval = jnp.zeros((16,), jnp.float32)

        def step(j, carry):
            k = j % 2

            @pl.when(j >= 2)
            def _drain_and_clear():
                pltpu.make_async_copy(
                    rows.at[k], y_hbm.at[base + j - 2, pl.ds(f0, FH)],
                    sems.at[k]).wait()
                for i in range(VPB):
                    old = idxs[k, pl.ds(16 * i, 16)]
                    plsc.store_scatter(rows.at[k], [fidx0 + 16 * i, old], zval)

            for i in range(VPB):
                lat, val = _enc16(xbuf[j, pl.ds(16 * i, 16)])
                idxs[k, pl.ds(16 * i, 16)] = lat
                plsc.store_scatter(rows.at[k], [fidx0 + 16 * i, lat], val)
            pltpu.async_copy(rows.at[k], y_hbm.at[base + j, pl.ds(f0, FH)],
                             sems.at[k])
            return carry

        lax.fori_loop(0, BPW, step, 0)
        for jj in (BPW - 2, BPW - 1):
            pltpu.make_async_copy(
                rows.at[jj % 2], y_hbm.at[base + jj, pl.ds(f0, FH)],
                sems.at[jj % 2]).wait()

    z = jnp.zeros((FH, NLANE), jnp.float32)
    y = enc(z, x)
    return y[:, :, :T]
